# baseline (device time: 15124 ns/iter reference)
import jax
import jax.numpy as jnp
from jax import lax
from jax.experimental import pallas as pl
from jax.experimental.pallas import tpu as pltpu

N_DEV = 4
B, SQ, SKV, HQ, DH = 2, 128, 512, 4, 64
D_MODEL = 512
D_QK = HQ * DH
CH = SKV // N_DEV
NBH = B * HQ


def kernel(x, Wq, K_ext, V_ext, Wo):
    q_all_host = jnp.dot(x.reshape(B * SQ, D_MODEL), Wq,
                         preferred_element_type=jnp.float32
                         ).astype(jnp.bfloat16)

    def body(q_ref, k_ref, v_ref, ctx_out,
             pay_send, pay_recv, ps_sems, pr_sems):
        my_pos = lax.axis_index("i")
        p_step = [jnp.bitwise_xor(my_pos, 1), 3 - my_pos]

        barrier_sem = pltpu.get_barrier_semaphore()
        for nbr in p_step:
            pl.semaphore_signal(barrier_sem, inc=1, device_id=(nbr,),
                                device_id_type=pl.DeviceIdType.MESH)
        pl.semaphore_wait(barrier_sem, 2)

        q_all = q_ref[...]
        k_loc = k_ref[...].astype(jnp.bfloat16).reshape(B * CH, D_QK)
        v_loc = v_ref[...].astype(jnp.bfloat16).reshape(B * CH, D_QK)

        kbg = my_pos * 2 + lax.broadcasted_iota(jnp.int32, (CH, SQ), 0) // 64
        qb = lax.broadcasted_iota(jnp.int32, (CH, SQ), 1) // 64
        mask = (qb == kbg) | (kbg == 0) | ((qb + kbg) % 3 == 0)

        ctx_blocks, m_blocks, l_blocks = [], [], []
        for b in range(B):
            for hh in range(HQ):
                q = q_all[b * SQ:(b + 1) * SQ, hh * DH:(hh + 1) * DH]
                kmat = k_loc[b * CH:(b + 1) * CH, hh * DH:(hh + 1) * DH]
                vmat = v_loc[b * CH:(b + 1) * CH, hh * DH:(hh + 1) * DH]
                s = lax.dot_general(
                    kmat, q, (((1,), (1,)), ((), ())),
                    preferred_element_type=jnp.float32) * 0.125
                s = jnp.where(mask, s, -1e9)
                m = jnp.max(s, axis=0, keepdims=True)
                w = jnp.exp(s - m)
                l = jnp.sum(w, axis=0, keepdims=True)
                ctx = lax.dot_general(
                    vmat, w.astype(jnp.bfloat16), (((0,), (0,)), ((), ())),
                    preferred_element_type=jnp.float32)
                ctx_blocks.append(ctx[None])
                m_blocks.append(m[None])
                l_blocks.append(l[None])
        ctx_acc = jnp.concatenate(ctx_blocks, axis=0)
        m_acc = jnp.concatenate(m_blocks, axis=0)
        l_acc = jnp.concatenate(l_blocks, axis=0)

        rdmas = []
        for s_i in range(2):
            pay_send[s_i, :, :DH, :] = ctx_acc.astype(jnp.bfloat16)
            pay_send[s_i, :, DH:DH + 1, :] = m_acc.astype(jnp.bfloat16)
            pay_send[s_i, :, DH + 1:DH + 2, :] = l_acc.astype(jnp.bfloat16)

            rdma = pltpu.make_async_remote_copy(
                src_ref=pay_send.at[s_i], dst_ref=pay_recv.at[s_i],
                send_sem=ps_sems.at[s_i], recv_sem=pr_sems.at[s_i],
                device_id=(p_step[s_i],),
                device_id_type=pl.DeviceIdType.MESH,
            )
            rdma.start()
            rdma.wait_recv()
            rdmas.append(rdma)

            ctx_o = pay_recv[s_i, :, :DH, :].astype(jnp.float32)
            m_o = pay_recv[s_i, :, DH:DH + 1, :].astype(jnp.float32)
            l_o = pay_recv[s_i, :, DH + 1:DH + 2, :].astype(jnp.float32)

            m_new = jnp.maximum(m_acc, m_o)
            alpha = jnp.exp(m_acc - m_new)
            beta = jnp.exp(m_o - m_new)
            ctx_acc = alpha * ctx_acc + beta * ctx_o
            l_acc = alpha * l_acc + beta * l_o
            m_acc = m_new

        ctx_out[...] = (ctx_acc / l_acc).astype(jnp.bfloat16)

        for rdma in rdmas:
            rdma.wait_send()

    ctx_n = pl.pallas_call(
        body,
        out_shape=jax.ShapeDtypeStruct((NBH, DH, SQ), jnp.bfloat16),
        in_specs=[pl.BlockSpec(memory_space=pltpu.VMEM)] * 3,
        out_specs=pl.BlockSpec(memory_space=pltpu.VMEM),
        scratch_shapes=[
            pltpu.VMEM((2, NBH, DH + 2, SQ), jnp.bfloat16),
            pltpu.VMEM((2, NBH, DH + 2, SQ), jnp.bfloat16),
            pltpu.SemaphoreType.DMA((2,)),
            pltpu.SemaphoreType.DMA((2,)),
        ],
        compiler_params=pltpu.CompilerParams(collective_id=0),
    )(q_all_host, K_ext, V_ext)

    return lax.dot_general(
        ctx_n.reshape(B, HQ, DH, SQ), Wo.reshape(HQ, DH, D_MODEL),
        (((1, 2), (0, 1)), ((), ())),
        preferred_element_type=jnp.float32)


# device time: 13377 ns/iter; 1.1306x vs baseline; 1.1306x over previous
import jax
import jax.numpy as jnp
from jax import lax
from jax.experimental import pallas as pl
from jax.experimental.pallas import tpu as pltpu

N_DEV = 4
B, SQ, SKV, HQ, DH = 2, 128, 512, 4, 64
D_MODEL = 512
D_QK = HQ * DH
CH = SKV // N_DEV
NBH = B * HQ


def kernel(x, Wq, K_ext, V_ext, Wo):
    def body(x_ref, wq_ref, k_ref, v_ref, wo_ref, out_ref,
             pay_send, pay_recv, ps_sems, pr_sems):
        my_pos = lax.axis_index("i")
        p_step = [jnp.bitwise_xor(my_pos, 1), 3 - my_pos]

        barrier_sem = pltpu.get_barrier_semaphore()
        for nbr in p_step:
            pl.semaphore_signal(barrier_sem, inc=1, device_id=(nbr,),
                                device_id_type=pl.DeviceIdType.MESH)
        pl.semaphore_wait(barrier_sem, 2)

        x_flat = x_ref[...].reshape(B * SQ, D_MODEL).astype(jnp.bfloat16)
        q_all = jnp.dot(x_flat, wq_ref[...].astype(jnp.bfloat16),
                        preferred_element_type=jnp.float32)

        k_loc = k_ref[...].astype(jnp.bfloat16).reshape(B * CH, D_QK)
        v_loc = v_ref[...].astype(jnp.bfloat16).reshape(B * CH, D_QK)

        kbg = my_pos * 2 + lax.broadcasted_iota(jnp.int32, (CH, SQ), 0) // 64
        qb = lax.broadcasted_iota(jnp.int32, (CH, SQ), 1) // 64
        mask = (qb == kbg) | (kbg == 0) | ((qb + kbg) % 3 == 0)

        def partial(blks):
            ctx_l, m_l, l_l = [], [], []
            for blk in blks:
                b, hh = blk // HQ, blk % HQ
                q = q_all[b * SQ:(b + 1) * SQ,
                          hh * DH:(hh + 1) * DH].astype(jnp.bfloat16)
                kmat = k_loc[b * CH:(b + 1) * CH, hh * DH:(hh + 1) * DH]
                vmat = v_loc[b * CH:(b + 1) * CH, hh * DH:(hh + 1) * DH]
                s = lax.dot_general(
                    kmat, q, (((1,), (1,)), ((), ())),
                    preferred_element_type=jnp.float32) * 0.125
                s = jnp.where(mask, s, -1e9)
                m = jnp.max(s, axis=0, keepdims=True)
                w = jnp.exp(s - m)
                l = jnp.sum(w, axis=0, keepdims=True)
                ctx = lax.dot_general(
                    vmat, w.astype(jnp.bfloat16), (((0,), (0,)), ((), ())),
                    preferred_element_type=jnp.float32)
                ctx_l.append(ctx[None])
                m_l.append(m[None])
                l_l.append(l[None])
            return (jnp.concatenate(ctx_l, axis=0),
                    jnp.concatenate(m_l, axis=0),
                    jnp.concatenate(l_l, axis=0))

        NH = NBH // 2

        def send_half(s_i, h_i, ctx, m, l):
            pay_send[s_i, h_i, :, :DH, :] = ctx.astype(jnp.bfloat16)
            pay_send[s_i, h_i, :, DH:DH + 1, :] = m.astype(jnp.bfloat16)
            pay_send[s_i, h_i, :, DH + 1:DH + 2, :] = l.astype(jnp.bfloat16)
            rdma = pltpu.make_async_remote_copy(
                src_ref=pay_send.at[s_i, h_i], dst_ref=pay_recv.at[s_i, h_i],
                send_sem=ps_sems.at[s_i, h_i], recv_sem=pr_sems.at[s_i, h_i],
                device_id=(p_step[s_i],),
                device_id_type=pl.DeviceIdType.MESH,
            )
            rdma.start()
            return rdma

        def combine(s_i, h_i, ctx, m, l):
            ctx_o = pay_recv[s_i, h_i, :, :DH, :].astype(jnp.float32)
            m_o = pay_recv[s_i, h_i, :, DH:DH + 1, :].astype(jnp.float32)
            l_o = pay_recv[s_i, h_i, :, DH + 1:DH + 2, :].astype(jnp.float32)
            m_new = jnp.maximum(m, m_o)
            alpha = jnp.exp(m - m_new)
            beta = jnp.exp(m_o - m_new)
            return (alpha * ctx + beta * ctx_o,
                    m_new,
                    alpha * l + beta * l_o)

        ctxA, mA, lA = partial(range(NH))
        rA0 = send_half(0, 0, ctxA, mA, lA)
        ctxB, mB, lB = partial(range(NH, NBH))
        rB0 = send_half(0, 1, ctxB, mB, lB)
        rA0.wait_recv()
        ctxA, mA, lA = combine(0, 0, ctxA, mA, lA)
        rA1 = send_half(1, 0, ctxA, mA, lA)
        rB0.wait_recv()
        ctxB, mB, lB = combine(0, 1, ctxB, mB, lB)
        rB1 = send_half(1, 1, ctxB, mB, lB)
        rA1.wait_recv()
        ctxA, mA, lA = combine(1, 0, ctxA, mA, lA)
        ctx_nA = (ctxA / lA).astype(jnp.bfloat16)
        rB1.wait_recv()
        ctxB, mB, lB = combine(1, 1, ctxB, mB, lB)
        rdmas = [rA0, rB0, rA1, rB1]

        ctx_n = jnp.concatenate(
            [ctx_nA, (ctxB / lB).astype(jnp.bfloat16)], axis=0)
        wo = wo_ref[...].astype(jnp.bfloat16)
        for b in range(B):
            acc = jnp.zeros((SQ, D_MODEL), jnp.float32)
            for hh in range(HQ):
                acc = acc + lax.dot_general(
                    ctx_n[b * HQ + hh], wo[hh * DH:(hh + 1) * DH, :],
                    (((0,), (0,)), ((), ())),
                    preferred_element_type=jnp.float32)
            out_ref[b, :, :] = acc.astype(jnp.bfloat16)

        for rdma in rdmas:
            rdma.wait_send()

    return pl.pallas_call(
        body,
        out_shape=jax.ShapeDtypeStruct((B, SQ, D_MODEL), jnp.bfloat16),
        in_specs=[pl.BlockSpec(memory_space=pltpu.VMEM)] * 5,
        out_specs=pl.BlockSpec(memory_space=pltpu.VMEM),
        scratch_shapes=[
            pltpu.VMEM((2, 2, NBH // 2, DH + 2, SQ), jnp.bfloat16),
            pltpu.VMEM((2, 2, NBH // 2, DH + 2, SQ), jnp.bfloat16),
            pltpu.SemaphoreType.DMA((2, 2)),
            pltpu.SemaphoreType.DMA((2, 2)),
        ],
        compiler_params=pltpu.CompilerParams(collective_id=0),
    )(x, Wq, K_ext, V_ext, Wo)


# device time: 12878 ns/iter; 1.1744x vs baseline; 1.0387x over previous
import jax
import jax.numpy as jnp
from jax import lax
from jax.experimental import pallas as pl
from jax.experimental.pallas import tpu as pltpu

N_DEV = 4
B, SQ, SKV, HQ, DH = 2, 128, 512, 4, 64
D_MODEL = 512
D_QK = HQ * DH
CH = SKV // N_DEV
NBH = B * HQ
PIECES = 4
NPP = NBH // PIECES


def kernel(x, Wq, K_ext, V_ext, Wo):
    def body(x_ref, wq_ref, k_ref, v_ref, wo_ref, out_ref,
             pay_send, pay_recv, ps_sems, pr_sems):
        my_pos = lax.axis_index("i")
        p_step = [jnp.bitwise_xor(my_pos, 1), 3 - my_pos]

        barrier_sem = pltpu.get_barrier_semaphore()
        for nbr in p_step:
            pl.semaphore_signal(barrier_sem, inc=1, device_id=(nbr,),
                                device_id_type=pl.DeviceIdType.MESH)
        pl.semaphore_wait(barrier_sem, 2)

        x_flat = x_ref[...].reshape(B * SQ, D_MODEL).astype(jnp.bfloat16)
        q_all = jnp.dot(x_flat, wq_ref[...].astype(jnp.bfloat16),
                        preferred_element_type=jnp.float32)

        k_loc = k_ref[...].astype(jnp.bfloat16).reshape(B * CH, D_QK)
        v_loc = v_ref[...].astype(jnp.bfloat16).reshape(B * CH, D_QK)

        kbg = my_pos * 2 + lax.broadcasted_iota(jnp.int32, (CH, SQ), 0) // 64
        qb = lax.broadcasted_iota(jnp.int32, (CH, SQ), 1) // 64
        mask = (qb == kbg) | (kbg == 0) | ((qb + kbg) % 3 == 0)

        def partial(blks):
            ctx_l, m_l, l_l = [], [], []
            for blk in blks:
                b, hh = blk // HQ, blk % HQ
                q = q_all[b * SQ:(b + 1) * SQ,
                          hh * DH:(hh + 1) * DH].astype(jnp.bfloat16)
                kmat = k_loc[b * CH:(b + 1) * CH, hh * DH:(hh + 1) * DH]
                vmat = v_loc[b * CH:(b + 1) * CH, hh * DH:(hh + 1) * DH]
                s = lax.dot_general(
                    kmat, q, (((1,), (1,)), ((), ())),
                    preferred_element_type=jnp.float32) * 0.125
                s = jnp.where(mask, s, -1e9)
                m = jnp.max(s, axis=0, keepdims=True)
                w = jnp.exp(s - m)
                l = jnp.sum(w, axis=0, keepdims=True)
                ctx = lax.dot_general(
                    vmat, w.astype(jnp.bfloat16), (((0,), (0,)), ((), ())),
                    preferred_element_type=jnp.float32)
                ctx_l.append(ctx[None])
                m_l.append(m[None])
                l_l.append(l[None])
            return (jnp.concatenate(ctx_l, axis=0),
                    jnp.concatenate(m_l, axis=0),
                    jnp.concatenate(l_l, axis=0))

        def send_piece(s_i, h_i, ctx, m, l):
            pay_send[s_i, h_i, :, :DH, :] = ctx.astype(jnp.bfloat16)
            pay_send[s_i, h_i, :, DH:DH + 1, :] = m.astype(jnp.bfloat16)
            pay_send[s_i, h_i, :, DH + 1:DH + 2, :] = l.astype(jnp.bfloat16)
            rdma = pltpu.make_async_remote_copy(
                src_ref=pay_send.at[s_i, h_i], dst_ref=pay_recv.at[s_i, h_i],
                send_sem=ps_sems.at[s_i, h_i], recv_sem=pr_sems.at[s_i, h_i],
                device_id=(p_step[s_i],),
                device_id_type=pl.DeviceIdType.MESH,
            )
            rdma.start()
            return rdma

        def combine(s_i, h_i, ctx, m, l):
            ctx_o = pay_recv[s_i, h_i, :, :DH, :].astype(jnp.float32)
            m_o = pay_recv[s_i, h_i, :, DH:DH + 1, :].astype(jnp.float32)
            l_o = pay_recv[s_i, h_i, :, DH + 1:DH + 2, :].astype(jnp.float32)
            m_new = jnp.maximum(m, m_o)
            alpha = jnp.exp(m - m_new)
            beta = jnp.exp(m_o - m_new)
            return (alpha * ctx + beta * ctx_o,
                    m_new,
                    alpha * l + beta * l_o)

        state, r0, r1, ctx_pieces = [], [], [], []
        for p in range(PIECES):
            st = partial(range(p * NPP, (p + 1) * NPP))
            r0.append(send_piece(0, p, *st))
            state.append(st)
        for p in range(PIECES):
            r0[p].wait_recv()
            st = combine(0, p, *state[p])
            r1.append(send_piece(1, p, *st))
            state[p] = st
        for p in range(PIECES):
            r1[p].wait_recv()
            ctx, _, l = combine(1, p, *state[p])
            ctx_pieces.append((ctx / l).astype(jnp.bfloat16))
        rdmas = r0 + r1

        ctx_n = jnp.concatenate(ctx_pieces, axis=0)
        wo = wo_ref[...].astype(jnp.bfloat16)
        for b in range(B):
            acc = jnp.zeros((SQ, D_MODEL), jnp.float32)
            for hh in range(HQ):
                acc = acc + lax.dot_general(
                    ctx_n[b * HQ + hh], wo[hh * DH:(hh + 1) * DH, :],
                    (((0,), (0,)), ((), ())),
                    preferred_element_type=jnp.float32)
            out_ref[b, :, :] = acc.astype(jnp.bfloat16)

        for rdma in rdmas:
            rdma.wait_send()

    return pl.pallas_call(
        body,
        out_shape=jax.ShapeDtypeStruct((B, SQ, D_MODEL), jnp.bfloat16),
        in_specs=[pl.BlockSpec(memory_space=pltpu.VMEM)] * 5,
        out_specs=pl.BlockSpec(memory_space=pltpu.VMEM),
        scratch_shapes=[
            pltpu.VMEM((2, PIECES, NPP, DH + 2, SQ), jnp.bfloat16),
            pltpu.VMEM((2, PIECES, NPP, DH + 2, SQ), jnp.bfloat16),
            pltpu.SemaphoreType.DMA((2, PIECES)),
            pltpu.SemaphoreType.DMA((2, PIECES)),
        ],
        compiler_params=pltpu.CompilerParams(collective_id=0),
    )(x, Wq, K_ext, V_ext, Wo)
